# Initial kernel scaffold; baseline (speedup 1.0000x reference)
import jax
import jax.numpy as jnp
from jax.experimental import pallas as pl


def kernel(Q, K, V, route_mat, ids, mask):
    raise NotImplementedError("write your pallas kernel here")


# speedup vs baseline: 1.1053x; 1.1053x over previous
import functools
import math

import jax
import jax.numpy as jnp
from jax.experimental import pallas as pl
from jax.experimental.pallas import tpu as pltpu


def _attn_body(ids_ref, q_ref, k_ref, v_ref, bias_ref, o_ref):
    q = q_ref[0, 0]
    k = k_ref[0, 0]
    v = v_ref[0, 0]
    d = q.shape[-1]
    s = jax.lax.dot_general(
        q, k, (((1,), (1,)), ((), ())), preferred_element_type=jnp.float32
    )
    s = s * (1.0 / math.sqrt(d)) + bias_ref[0]
    m = jnp.max(s, axis=-1, keepdims=True)
    e = jnp.exp(s - m)
    p = e / jnp.sum(e, axis=-1, keepdims=True)
    o_ref[0, 0] = jax.lax.dot_general(
        p, v, (((1,), (0,)), ((), ())), preferred_element_type=jnp.float32
    )


def kernel(Q, K, V, route_mat, ids, mask):
    B, H, S, D = Q.shape
    E, cap = ids.shape
    Bp = E * cap
    flat = ids.reshape(-1).astype(jnp.int32)
    bias = ((mask - 1.0) * 1000000.0).reshape(B, 1, S)

    BQ = min(512, S)
    grid = (Bp, H, S // BQ)

    out = pl.pallas_call(
        _attn_body,
        grid_spec=pltpu.PrefetchScalarGridSpec(
            num_scalar_prefetch=1,
            grid=grid,
            in_specs=[
                pl.BlockSpec((1, 1, BQ, D), lambda b, h, qi, ids_ref: (ids_ref[b], h, qi, 0)),
                pl.BlockSpec((1, 1, S, D), lambda b, h, qi, ids_ref: (ids_ref[b], h, 0, 0)),
                pl.BlockSpec((1, 1, S, D), lambda b, h, qi, ids_ref: (ids_ref[b], h, 0, 0)),
                pl.BlockSpec((1, 1, S), lambda b, h, qi, ids_ref: (ids_ref[b], 0, 0)),
            ],
            out_specs=pl.BlockSpec((1, 1, BQ, D), lambda b, h, qi, ids_ref: (b, h, qi, 0)),
        ),
        out_shape=jax.ShapeDtypeStruct((Bp, H, S, D), jnp.float32),
    )(flat, Q, K, V, bias)
    return out.reshape(E, cap, H, S, D)
